# baseline (speedup 1.0000x reference)
import jax
import jax.numpy as jnp
from jax import lax
from jax.experimental import pallas as pl
from jax.experimental.pallas import tpu as pltpu
from jax.experimental.pallas import tpu_sc as plsc

VOCAB = 100000
D = 128
B = 1024
L = 200
M1 = 104
M2 = L - M1

NC = 2
NS = 16
NW = NC * NS

CHUNK = 128
NCHUNK = B // CHUNK
NRING = 4


def _make_pool(lsub, lout=None):
    lout = lsub if lout is None else lout
    lpw = -(-lsub // NW)
    lo = lsub // NW
    nheavy = lsub - NW * lo

    def body(table_hbm, idx_hbm, out_hbm, idx_v, rows_v, acc_v,
             isem, sem0, sem1, sem2, sem3):
        wid = lax.axis_index("s") * NC + lax.axis_index("c")
        start = jnp.where(wid < nheavy, wid * (lo + 1),
                          nheavy * (lo + 1) + (wid - nheavy) * lo)
        n = jnp.where(wid < nheavy, lo + 1, lo)
        sems = (sem0, sem1, sem2, sem3)

        pltpu.sync_copy(idx_hbm.at[start], idx_v.at[0])
        for j in range(lpw):
            l = start + j

            @pl.when(j < n)
            def _():
                ib = j % 2
                if j + 1 < lpw:
                    @pl.when(j + 1 < n)
                    def _():
                        pltpu.async_copy(
                            idx_hbm.at[l + 1], idx_v.at[(j + 1) % 2], isem)
                acc = tuple(jnp.zeros((16,), jnp.float32) for _ in range(8))
                cps = [None] * NCHUNK
                for c in range(NRING - 1):
                    cps[c] = pltpu.async_copy(
                        table_hbm.at[idx_v.at[ib, c]], rows_v.at[c % NRING],
                        sems[c % NRING])
                for c in range(NCHUNK):
                    if c + NRING - 1 < NCHUNK:
                        nb = (c + NRING - 1) % NRING
                        cps[c + NRING - 1] = pltpu.async_copy(
                            table_hbm.at[idx_v.at[ib, c + NRING - 1]],
                            rows_v.at[nb], sems[nb])
                    cps[c].wait()
                    buf = c % NRING

                    def rbody(r, a):
                        return tuple(
                            a[k] + rows_v[buf, r, k * 16:(k + 1) * 16]
                            for k in range(8)
                        )

                    acc = lax.fori_loop(0, CHUNK, rbody, acc, unroll=8)
                for k in range(8):
                    acc_v[k * 16:(k + 1) * 16] = acc[k] * (1.0 / B)
                pltpu.sync_copy(acc_v, out_hbm.at[l])
                if j + 1 < lpw:
                    @pl.when(j + 1 < n)
                    def _():
                        pltpu.make_async_copy(
                            idx_hbm.at[l + 1], idx_v.at[(j + 1) % 2],
                            isem).wait()

    mesh = plsc.VectorSubcoreMesh(core_axis_name="c", subcore_axis_name="s")
    return pl.kernel(
        body,
        mesh=mesh,
        out_type=jax.ShapeDtypeStruct((lout, D), jnp.float32),
        scratch_types=[
            pltpu.VMEM((2, NCHUNK, CHUNK), jnp.int32),
            pltpu.VMEM((NRING, CHUNK, D), jnp.float32),
            pltpu.VMEM((D,), jnp.float32),
            pltpu.SemaphoreType.DMA,
            pltpu.SemaphoreType.DMA,
            pltpu.SemaphoreType.DMA,
            pltpu.SemaphoreType.DMA,
            pltpu.SemaphoreType.DMA,
        ],
    )


_pool_1 = _make_pool(M1)
_pool_2 = _make_pool(M2, lout=M1)


NBLK = 16384
GRID = -(-VOCAB // NBLK)


def _mm_body(x_ref, w_ref, b_ref, o_ref):
    x = x_ref[...].astype(jnp.bfloat16)
    w = w_ref[...].astype(jnp.bfloat16)
    o_ref[...] = lax.dot_general(
        x, w,
        (((1,), (1,)), ((), ())),
        preferred_element_type=jnp.float32,
    ) + b_ref[...]


def _mm_body_alias(x_ref, w_ref, b_ref, prev_ref, o_ref):
    del prev_ref
    _mm_body(x_ref, w_ref, b_ref, o_ref)


def _tc_project_1(pooled, W, b2d):
    return pl.pallas_call(
        _mm_body,
        grid=(GRID,),
        in_specs=[
            pl.BlockSpec((M1, D), lambda i: (0, 0)),
            pl.BlockSpec((NBLK, D), lambda i: (i, 0)),
            pl.BlockSpec((1, NBLK), lambda i: (0, i)),
        ],
        out_specs=pl.BlockSpec((M1, NBLK), lambda i: (0, i)),
        out_shape=jax.ShapeDtypeStruct((L, VOCAB), jnp.float32),
    )(pooled, W, b2d)


def _tc_project_2(pooled, W, b2d, prev):
    return pl.pallas_call(
        _mm_body_alias,
        grid=(GRID,),
        in_specs=[
            pl.BlockSpec((M1, D), lambda i: (0, 0)),
            pl.BlockSpec((NBLK, D), lambda i: (i, 0)),
            pl.BlockSpec((1, NBLK), lambda i: (0, i)),
            pl.BlockSpec(memory_space=pl.ANY),
        ],
        out_specs=pl.BlockSpec((M1, NBLK), lambda i: (1, i)),
        out_shape=jax.ShapeDtypeStruct((L, VOCAB), jnp.float32),
        input_output_aliases={3: 0},
    )(pooled, W, b2d, prev)


def kernel(context_idxs, emb_table, W, b):
    idx3 = context_idxs.T.reshape(L, NCHUNK, CHUNK).astype(jnp.int32)
    b2d = b.reshape(1, VOCAB)
    pooled_1 = _pool_1(emb_table, idx3[:M1])
    pooled_2 = _pool_2(emb_table, idx3[M1:])
    out_1 = _tc_project_1(pooled_1, W, b2d)
    return _tc_project_2(pooled_2, W, b2d, out_1)


# speedup vs baseline: 1.0514x; 1.0514x over previous
import jax
import jax.numpy as jnp
from jax import lax
from jax.experimental import pallas as pl
from jax.experimental.pallas import tpu as pltpu
from jax.experimental.pallas import tpu_sc as plsc

VOCAB = 100000
D = 128
B = 1024
L = 200

NC = 2
NS = 16
NW = NC * NS

CHUNK = 128
NCHUNK = B // CHUNK
NRING = 6


def _make_pool(lsub, lout=None):
    lout = lsub if lout is None else lout
    lpw = -(-lsub // NW)
    lo = lsub // NW
    nheavy = lsub - NW * lo

    def body(table_hbm, idx_hbm, out_hbm, idx_v, rows_v, acc_v,
             isem, sem0, sem1, sem2, sem3, sem4, sem5):
        wid = lax.axis_index("s") * NC + lax.axis_index("c")
        start = jnp.where(wid < nheavy, wid * (lo + 1),
                          nheavy * (lo + 1) + (wid - nheavy) * lo)
        n = jnp.where(wid < nheavy, lo + 1, lo)
        sems = (sem0, sem1, sem2, sem3, sem4, sem5)

        pltpu.sync_copy(idx_hbm.at[start], idx_v.at[0])
        for j in range(lpw):
            l = start + j

            @pl.when(j < n)
            def _():
                ib = j % 2
                if j + 1 < lpw:
                    @pl.when(j + 1 < n)
                    def _():
                        pltpu.async_copy(
                            idx_hbm.at[l + 1], idx_v.at[(j + 1) % 2], isem)
                acc = tuple(jnp.zeros((16,), jnp.float32) for _ in range(8))
                cps = [None] * NCHUNK
                for c in range(NRING - 1):
                    cps[c] = pltpu.async_copy(
                        table_hbm.at[idx_v.at[ib, c]], rows_v.at[c % NRING],
                        sems[c % NRING])
                for c in range(NCHUNK):
                    if c + NRING - 1 < NCHUNK:
                        nb = (c + NRING - 1) % NRING
                        cps[c + NRING - 1] = pltpu.async_copy(
                            table_hbm.at[idx_v.at[ib, c + NRING - 1]],
                            rows_v.at[nb], sems[nb])
                    cps[c].wait()
                    buf = c % NRING

                    def rbody(r, a):
                        return tuple(
                            a[k] + rows_v[buf, r, k * 16:(k + 1) * 16]
                            for k in range(8)
                        )

                    acc = lax.fori_loop(0, CHUNK, rbody, acc, unroll=8)
                for k in range(8):
                    acc_v[k * 16:(k + 1) * 16] = acc[k] * (1.0 / B)
                pltpu.sync_copy(acc_v, out_hbm.at[l])
                if j + 1 < lpw:
                    @pl.when(j + 1 < n)
                    def _():
                        pltpu.make_async_copy(
                            idx_hbm.at[l + 1], idx_v.at[(j + 1) % 2],
                            isem).wait()

    mesh = plsc.VectorSubcoreMesh(core_axis_name="c", subcore_axis_name="s")
    return pl.kernel(
        body,
        mesh=mesh,
        out_type=jax.ShapeDtypeStruct((lout, D), jnp.float32),
        scratch_types=[
            pltpu.VMEM((2, NCHUNK, CHUNK), jnp.int32),
            pltpu.VMEM((NRING, CHUNK, D), jnp.float32),
            pltpu.VMEM((D,), jnp.float32),
            pltpu.SemaphoreType.DMA,
            pltpu.SemaphoreType.DMA,
            pltpu.SemaphoreType.DMA,
            pltpu.SemaphoreType.DMA,
            pltpu.SemaphoreType.DMA,
            pltpu.SemaphoreType.DMA,
            pltpu.SemaphoreType.DMA,
        ],
    )


_pool_all = _make_pool(L)


NBLK = 16384
GRID = -(-VOCAB // NBLK)


def _mm_body(x_ref, w_ref, b_ref, o_ref):
    x = x_ref[...].astype(jnp.bfloat16)
    w = w_ref[...].astype(jnp.bfloat16)
    o_ref[...] = lax.dot_general(
        x, w,
        (((1,), (1,)), ((), ())),
        preferred_element_type=jnp.float32,
    ) + b_ref[...]


def _tc_project(pooled, W, b2d):
    return pl.pallas_call(
        _mm_body,
        grid=(GRID,),
        in_specs=[
            pl.BlockSpec((L, D), lambda i: (0, 0)),
            pl.BlockSpec((NBLK, D), lambda i: (i, 0)),
            pl.BlockSpec((1, NBLK), lambda i: (0, i)),
        ],
        out_specs=pl.BlockSpec((L, NBLK), lambda i: (0, i)),
        out_shape=jax.ShapeDtypeStruct((L, VOCAB), jnp.float32),
    )(pooled, W, b2d)


def kernel(context_idxs, emb_table, W, b):
    idx3 = context_idxs.T.reshape(L, NCHUNK, CHUNK).astype(jnp.int32)
    b2d = b.reshape(1, VOCAB)
    pooled = _pool_all(emb_table, idx3)
    return _tc_project(pooled, W, b2d)


# speedup vs baseline: 1.0960x; 1.0424x over previous
import jax
import jax.numpy as jnp
from jax import lax
from jax.experimental import pallas as pl
from jax.experimental.pallas import tpu as pltpu
from jax.experimental.pallas import tpu_sc as plsc

VOCAB = 100000
D = 128
B = 1024
L = 200

NC = 2
NS = 16
NW = NC * NS

CHUNK = 128
NCHUNK = B // CHUNK
NRING = 4


def _make_pool(lsub, lout=None):
    lout = lsub if lout is None else lout
    lpw = -(-lsub // NW)
    lo = lsub // NW
    nheavy = lsub - NW * lo

    def body(table_hbm, idx_hbm, out_hbm, idx_v, rows_v, acc_v,
             isem, sem0, sem1, sem2, sem3):
        wid = lax.axis_index("s") * NC + lax.axis_index("c")
        start = jnp.where(wid < nheavy, wid * (lo + 1),
                          nheavy * (lo + 1) + (wid - nheavy) * lo)
        n = jnp.where(wid < nheavy, lo + 1, lo)
        sems = (sem0, sem1, sem2, sem3)

        pltpu.sync_copy(idx_hbm.at[start], idx_v.at[0])
        for j in range(lpw):
            l = start + j

            @pl.when(j < n)
            def _():
                ib = j % 2
                if j + 1 < lpw:
                    @pl.when(j + 1 < n)
                    def _():
                        pltpu.async_copy(
                            idx_hbm.at[l + 1], idx_v.at[(j + 1) % 2], isem)
                acc = tuple(jnp.zeros((16,), jnp.float32) for _ in range(8))
                cps = [None] * NCHUNK
                for c in range(NRING - 1):
                    cps[c] = pltpu.async_copy(
                        table_hbm.at[idx_v.at[ib, c]], rows_v.at[c % NRING],
                        sems[c % NRING])
                for c in range(NCHUNK):
                    if c + NRING - 1 < NCHUNK:
                        nb = (c + NRING - 1) % NRING
                        cps[c + NRING - 1] = pltpu.async_copy(
                            table_hbm.at[idx_v.at[ib, c + NRING - 1]],
                            rows_v.at[nb], sems[nb])
                    cps[c].wait()
                    buf = c % NRING

                    def rbody(r, a):
                        return tuple(
                            a[k] + rows_v[buf, r, k * 16:(k + 1) * 16]
                            for k in range(8)
                        )

                    acc = lax.fori_loop(0, CHUNK, rbody, acc, unroll=8)
                for k in range(8):
                    acc_v[k * 16:(k + 1) * 16] = acc[k] * (1.0 / B)
                pltpu.sync_copy(acc_v, out_hbm.at[l])
                if j + 1 < lpw:
                    @pl.when(j + 1 < n)
                    def _():
                        pltpu.make_async_copy(
                            idx_hbm.at[l + 1], idx_v.at[(j + 1) % 2],
                            isem).wait()

    mesh = plsc.VectorSubcoreMesh(core_axis_name="c", subcore_axis_name="s")
    return pl.kernel(
        body,
        mesh=mesh,
        out_type=jax.ShapeDtypeStruct((lout, D), jnp.float32),
        scratch_types=[
            pltpu.VMEM((2, NCHUNK, CHUNK), jnp.int32),
            pltpu.VMEM((NRING, CHUNK, D), jnp.float32),
            pltpu.VMEM((D,), jnp.float32),
            pltpu.SemaphoreType.DMA,
            pltpu.SemaphoreType.DMA,
            pltpu.SemaphoreType.DMA,
            pltpu.SemaphoreType.DMA,
            pltpu.SemaphoreType.DMA,
        ],
    )


_pool_all = _make_pool(L)


NBLK = 16384
GRID = -(-VOCAB // NBLK)


def _mm_body(x_ref, w_ref, b_ref, o_ref):
    x = x_ref[...].astype(jnp.bfloat16)
    w = w_ref[...].astype(jnp.bfloat16)
    o_ref[...] = lax.dot_general(
        x, w,
        (((1,), (1,)), ((), ())),
        preferred_element_type=jnp.float32,
    ) + b_ref[...]


def _tc_project(pooled, W, b2d):
    return pl.pallas_call(
        _mm_body,
        grid=(GRID,),
        in_specs=[
            pl.BlockSpec((L, D), lambda i: (0, 0)),
            pl.BlockSpec((NBLK, D), lambda i: (i, 0)),
            pl.BlockSpec((1, NBLK), lambda i: (0, i)),
        ],
        out_specs=pl.BlockSpec((L, NBLK), lambda i: (0, i)),
        out_shape=jax.ShapeDtypeStruct((L, VOCAB), jnp.float32),
    )(pooled, W, b2d)


def kernel(context_idxs, emb_table, W, b):
    idx3 = context_idxs.T.reshape(L, NCHUNK, CHUNK).astype(jnp.int32)
    b2d = b.reshape(1, VOCAB)
    pooled = _pool_all(emb_table, idx3)
    return _tc_project(pooled, W, b2d)


# speedup vs baseline: 1.1042x; 1.0075x over previous
import jax
import jax.numpy as jnp
from jax import lax
from jax.experimental import pallas as pl
from jax.experimental.pallas import tpu as pltpu
from jax.experimental.pallas import tpu_sc as plsc

VOCAB = 100000
D = 128
B = 1024
L = 200

NC = 2
NS = 16
NW = NC * NS

CHUNK = 128
NCHUNK = B // CHUNK
NRING = 4


def _make_pool(lsub, lout=None):
    lout = lsub if lout is None else lout
    lpw = -(-lsub // NW)
    lo = lsub // NW
    nheavy = lsub - NW * lo

    def body(table_hbm, idx_hbm, out_hbm, idx_v, rows_v, acc_v,
             isem, sem0, sem1, sem2, sem3):
        wid = lax.axis_index("s") * NC + lax.axis_index("c")
        start = jnp.where(wid < nheavy, wid * (lo + 1),
                          nheavy * (lo + 1) + (wid - nheavy) * lo)
        n = jnp.where(wid < nheavy, lo + 1, lo)
        sems = (sem0, sem1, sem2, sem3)

        pltpu.sync_copy(idx_hbm.at[start], idx_v.at[0])
        for j in range(lpw):
            l = start + j

            @pl.when(j < n)
            def _():
                ib = j % 2
                if j + 1 < lpw:
                    @pl.when(j + 1 < n)
                    def _():
                        pltpu.async_copy(
                            idx_hbm.at[l + 1], idx_v.at[(j + 1) % 2], isem)
                acc = tuple(jnp.zeros((16,), jnp.float32) for _ in range(8))
                cps = [None] * NCHUNK
                for c in range(NRING - 1):
                    cps[c] = pltpu.async_copy(
                        table_hbm.at[idx_v.at[ib, c]], rows_v.at[c % NRING],
                        sems[c % NRING])
                for c in range(NCHUNK):
                    if c + NRING - 1 < NCHUNK:
                        nb = (c + NRING - 1) % NRING
                        cps[c + NRING - 1] = pltpu.async_copy(
                            table_hbm.at[idx_v.at[ib, c + NRING - 1]],
                            rows_v.at[nb], sems[nb])
                    cps[c].wait()
                    buf = c % NRING

                    def rbody(r, a):
                        return tuple(
                            a[k] + rows_v[buf, r, k * 16:(k + 1) * 16]
                            for k in range(8)
                        )

                    acc = lax.fori_loop(0, CHUNK, rbody, acc, unroll=8)
                for k in range(8):
                    acc_v[k * 16:(k + 1) * 16] = acc[k] * (1.0 / B)
                pltpu.sync_copy(acc_v, out_hbm.at[l])
                if j + 1 < lpw:
                    @pl.when(j + 1 < n)
                    def _():
                        pltpu.make_async_copy(
                            idx_hbm.at[l + 1], idx_v.at[(j + 1) % 2],
                            isem).wait()

    mesh = plsc.VectorSubcoreMesh(core_axis_name="c", subcore_axis_name="s")
    return pl.kernel(
        body,
        mesh=mesh,
        out_type=jax.ShapeDtypeStruct((lout, D), jnp.float32),
        scratch_types=[
            pltpu.VMEM((2, NCHUNK, CHUNK), jnp.int32),
            pltpu.VMEM((NRING, CHUNK, D), jnp.float32),
            pltpu.VMEM((D,), jnp.float32),
            pltpu.SemaphoreType.DMA,
            pltpu.SemaphoreType.DMA,
            pltpu.SemaphoreType.DMA,
            pltpu.SemaphoreType.DMA,
            pltpu.SemaphoreType.DMA,
        ],
    )


_pool_all = _make_pool(L)


NBLK = 25088
GRID = -(-VOCAB // NBLK)


def _mm_body(x_ref, w_ref, b_ref, o_ref):
    x = x_ref[...].astype(jnp.bfloat16)
    w = w_ref[...].astype(jnp.bfloat16)
    o_ref[...] = lax.dot_general(
        x, w,
        (((1,), (1,)), ((), ())),
        preferred_element_type=jnp.float32,
    ) + b_ref[...]


def _tc_project(pooled, W, b2d):
    return pl.pallas_call(
        _mm_body,
        grid=(GRID,),
        compiler_params=pltpu.CompilerParams(
            vmem_limit_bytes=112 * 1024 * 1024),
        in_specs=[
            pl.BlockSpec((L, D), lambda i: (0, 0)),
            pl.BlockSpec((NBLK, D), lambda i: (i, 0)),
            pl.BlockSpec((1, NBLK), lambda i: (0, i)),
        ],
        out_specs=pl.BlockSpec((L, NBLK), lambda i: (0, i)),
        out_shape=jax.ShapeDtypeStruct((L, VOCAB), jnp.float32),
    )(pooled, W, b2d)


def kernel(context_idxs, emb_table, W, b):
    idx3 = context_idxs.T.reshape(L, NCHUNK, CHUNK).astype(jnp.int32)
    b2d = b.reshape(1, VOCAB)
    pooled = _pool_all(emb_table, idx3)
    return _tc_project(pooled, W, b2d)
